# baseline (speedup 1.0000x reference)
import functools

import jax
import jax.numpy as jnp
from jax import lax
from jax.experimental import pallas as pl
from jax.experimental.pallas import tpu as pltpu
from jax.experimental.pallas import tpu_sc as plsc

N_PROP = 4096
N_PAIRS = 16384
INPUT_DIM = 512
EMBED_DIM = 300
GEO_DIM = 128
HIDDEN = 512
NUM_OBJ = 150
NUM_REL = 50

NC = 2
NS = 16
NW = NC * NS
MAT = N_PROP * N_PROP
PAD = NW * 1024

_f32 = jnp.float32


def _prep_body(boxes, labels, table, w1, b1, w2, b2,
               gw_ps, gw_ss, gw_pd, gw_sd, a_out, b_out):
    b = boxes[...]
    x1 = b[:, 0:1]
    y1 = b[:, 1:2]
    x2 = b[:, 2:3]
    y2 = b[:, 3:4]
    w = x2 - x1 + 1.0
    h = y2 - y1 + 1.0
    x = x1 + 0.5 * w
    y = y1 + 0.5 * h
    cols = (w, h, x, y, x1, y1, x2, y2, w * h)
    acc = b1[...]
    for j, cj in enumerate(cols):
        acc = acc + cj * w1[j:j + 1, :]
    p1 = jnp.maximum(acc, 0.0)
    pos = jnp.dot(p1, w2[...], preferred_element_type=_f32) + b2[...]
    lab = labels[...]
    onehot = (lab == lax.broadcasted_iota(jnp.int32, (lab.shape[0], NUM_OBJ), 1)
              ).astype(_f32)
    sem = jnp.dot(onehot, table[...], preferred_element_type=_f32)
    rp = jnp.maximum(pos, 0.0)
    rs = jnp.maximum(sem, 0.0)
    a_f = (jnp.dot(rp, gw_ps[...], preferred_element_type=_f32) +
           jnp.dot(rs, gw_ss[...], preferred_element_type=_f32))
    b_f = (jnp.dot(rp, gw_pd[...], preferred_element_type=_f32) +
           jnp.dot(rs, gw_sd[...], preferred_element_type=_f32))
    a_out[...] = _pack_halves(a_f)
    b_out[...] = _pack_halves(b_f)


def _pack_halves(x):
    lo = lax.bitcast_convert_type(x[:, :HIDDEN // 2].astype(jnp.bfloat16),
                                  jnp.uint16).astype(jnp.uint32)
    hi = lax.bitcast_convert_type(x[:, HIDDEN // 2:].astype(jnp.bfloat16),
                                  jnp.uint16).astype(jnp.uint32)
    return lax.bitcast_convert_type((hi << 16) | lo, jnp.int32)


def _unpack_halves(w):
    u = lax.bitcast_convert_type(w, jnp.uint32)
    lo = lax.bitcast_convert_type((u & 0xFFFF).astype(jnp.uint16),
                                  jnp.bfloat16).astype(_f32)
    hi = lax.bitcast_convert_type((u >> 16).astype(jnp.uint16),
                                  jnp.bfloat16).astype(_f32)
    return jnp.concatenate([lo, hi], axis=1)


def _prep_call(boxes, labels, table, w1, b1, w2, b2, gw_ps, gw_ss, gw_pd, gw_sd):
    blk = 1024
    n_blk = N_PROP // blk
    row = lambda i: (i, 0)
    rep = lambda i: (0, 0)
    return pl.pallas_call(
        _prep_body,
        grid=(n_blk,),
        in_specs=[
            pl.BlockSpec((blk, 4), row),
            pl.BlockSpec((blk, 1), row),
            pl.BlockSpec((NUM_OBJ, EMBED_DIM), rep),
            pl.BlockSpec((9, GEO_DIM), rep),
            pl.BlockSpec((1, GEO_DIM), rep),
            pl.BlockSpec((GEO_DIM, GEO_DIM), rep),
            pl.BlockSpec((1, GEO_DIM), rep),
            pl.BlockSpec((GEO_DIM, HIDDEN), rep),
            pl.BlockSpec((EMBED_DIM, HIDDEN), rep),
            pl.BlockSpec((GEO_DIM, HIDDEN), rep),
            pl.BlockSpec((EMBED_DIM, HIDDEN), rep),
        ],
        out_specs=[pl.BlockSpec((blk, HIDDEN // 2), row),
                   pl.BlockSpec((blk, HIDDEN // 2), row)],
        out_shape=[jax.ShapeDtypeStruct((N_PROP, HIDDEN // 2), jnp.int32)] * 2,
    )(boxes, labels, table, w1, b1, w2, b2, gw_ps, gw_ss, gw_pd, gw_sd)


def _gather_body(src_hbm, dst_hbm, a_hbm, b_hbm, gs_hbm, gd_hbm,
                 idx0, idx1, rows0, rows1, is0, is1, os0, os1):
    wid = lax.axis_index("s") * NC + lax.axis_index("c")
    base = wid * (N_PAIRS // NW)
    idx_bufs = (idx0, idx1)
    row_bufs = (rows0, rows1)
    isems = (is0, is1)
    osems = (os0, os1)
    steps = ([(src_hbm, a_hbm, gs_hbm, j) for j in range(4)] +
             [(dst_hbm, b_hbm, gd_hbm, j) for j in range(4)])
    in_d = [None, None]
    out_d = [None, None]

    def fire(t):
        bi = t % 2
        if out_d[bi] is not None:
            out_d[bi].wait()
            out_d[bi] = None
        isrc, tbl, _, j = steps[t]
        pltpu.sync_copy(isrc.at[pl.ds(base + j * 128, 128)], idx_bufs[bi])
        in_d[bi] = pltpu.async_copy(tbl.at[idx_bufs[bi]], row_bufs[bi],
                                    isems[bi])

    fire(0)
    for t in range(len(steps)):
        if t + 1 < len(steps):
            fire(t + 1)
        bi = t % 2
        in_d[bi].wait()
        _, _, out, j = steps[t]
        out_d[bi] = pltpu.async_copy(
            row_bufs[bi], out.at[pl.ds(base + j * 128, 128)], osems[bi])
    for bi in range(2):
        if out_d[bi] is not None:
            out_d[bi].wait()


def _gather_call(src, dst, a, b):
    mesh = plsc.VectorSubcoreMesh(core_axis_name="c", subcore_axis_name="s",
                                  num_cores=NC, num_subcores=NS)
    f = functools.partial(
        pl.kernel, _gather_body,
        out_type=[jax.ShapeDtypeStruct((N_PAIRS, HIDDEN // 2), jnp.int32)] * 2,
        mesh=mesh,
        scratch_types=[
            pltpu.VMEM((128,), jnp.int32),
            pltpu.VMEM((128,), jnp.int32),
            pltpu.VMEM((128, HIDDEN // 2), jnp.int32),
            pltpu.VMEM((128, HIDDEN // 2), jnp.int32),
            pltpu.SemaphoreType.DMA,
            pltpu.SemaphoreType.DMA,
            pltpu.SemaphoreType.DMA,
            pltpu.SemaphoreType.DMA,
        ],
    )()
    return f(src, dst, a, b)


def _vis_body(vis, vw, vb, out):
    out[...] = (jnp.dot(jnp.maximum(vis[...], 0.0).astype(jnp.bfloat16),
                        vw[...], preferred_element_type=_f32) + vb[...]
                ).astype(jnp.bfloat16)


def _vis_call(vis, vw, vb):
    blk = 2048
    row = lambda i: (i, 0)
    rep = lambda i: (0, 0)
    return pl.pallas_call(
        _vis_body,
        grid=(N_PAIRS // blk,),
        in_specs=[
            pl.BlockSpec((blk, INPUT_DIM), row),
            pl.BlockSpec((INPUT_DIM, HIDDEN), rep),
            pl.BlockSpec((1, HIDDEN), rep),
        ],
        out_specs=pl.BlockSpec((blk, HIDDEN), row),
        out_shape=jax.ShapeDtypeStruct((N_PAIRS, HIDDEN), jnp.bfloat16),
    )(vis, vw, vb)


def _main_body(v16, gs, gd, gb, fgv, fgg, fbv, fbg, fwv, fwg, fb,
               cg, cb_ln, cw, cb, hw, hb, out_logits, out_scores):
    v = v16[...].astype(_f32)
    g = _unpack_halves(gs[...]) + _unpack_halves(gd[...]) + gb[...]
    width = 2.0 * HIDDEN
    mu = (jnp.sum(v, axis=1, keepdims=True) +
          jnp.sum(g, axis=1, keepdims=True)) / width
    dv = v - mu
    dg = g - mu
    var = (jnp.sum(dv * dv, axis=1, keepdims=True) +
           jnp.sum(dg * dg, axis=1, keepdims=True)) / width
    inv = lax.rsqrt(var + 1e-5)
    nv = jnp.maximum(dv * inv * fgv[...] + fbv[...], 0.0).astype(jnp.bfloat16)
    ng = jnp.maximum(dg * inv * fgg[...] + fbg[...], 0.0).astype(jnp.bfloat16)
    h = (jnp.dot(nv, fwv[...], preferred_element_type=_f32) +
         jnp.dot(ng, fwg[...], preferred_element_type=_f32) + fb[...])
    mu2 = jnp.mean(h, axis=1, keepdims=True)
    dh = h - mu2
    var2 = jnp.mean(dh * dh, axis=1, keepdims=True)
    n2 = jnp.maximum(dh * lax.rsqrt(var2 + 1e-5) * cg[...] + cb_ln[...], 0.0)
    logits = jnp.dot(n2, cw[...], preferred_element_type=_f32) + cb[...]
    bin_l = jnp.dot(logits, hw[...], preferred_element_type=_f32) + hb[...]
    out_logits[...] = jnp.concatenate([logits, bin_l], axis=1)
    out_scores[...] = jax.nn.sigmoid(bin_l)


def _main_call(v16, gs, gd, gb, fgv, fgg, fbv, fbg, fwv, fwg, fb,
               cg, cb_ln, cw, cb, hw, hb):
    blk = 1024
    n_blk = N_PAIRS // blk
    row = lambda i: (i, 0)
    rep = lambda i: (0, 0)
    wide = lambda shape: pl.BlockSpec(shape, rep)
    return pl.pallas_call(
        _main_body,
        grid=(n_blk,),
        in_specs=[
            pl.BlockSpec((blk, HIDDEN), row),
            pl.BlockSpec((blk, HIDDEN // 2), row),
            pl.BlockSpec((blk, HIDDEN // 2), row),
            wide((1, HIDDEN)),
            wide((1, HIDDEN)),
            wide((1, HIDDEN)),
            wide((1, HIDDEN)),
            wide((1, HIDDEN)),
            wide((HIDDEN, HIDDEN)),
            wide((HIDDEN, HIDDEN)),
            wide((1, HIDDEN)),
            wide((1, HIDDEN)),
            wide((1, HIDDEN)),
            wide((HIDDEN, NUM_REL)),
            wide((1, NUM_REL)),
            wide((NUM_REL, 1)),
            wide((1, 1)),
        ],
        out_specs=[pl.BlockSpec((blk, NUM_REL + 1), row),
                   pl.BlockSpec((blk, 1), row)],
        out_shape=[jax.ShapeDtypeStruct((N_PAIRS, NUM_REL + 1), _f32),
                   jax.ShapeDtypeStruct((N_PAIRS, 1), _f32)],
    )(v16, gs, gd, gb, fgv, fgg, fbv, fbg, fwv, fwg, fb,
      cg, cb_ln, cw, cb, hw, hb)


_TILE_Z = MAT // NW
_ZBUF = 32768
_ROWS_PER_TILE = 4


def _zero_body(out_hbm, zbuf, sem):
    wid = lax.axis_index("s") * NC + lax.axis_index("c")

    def _fill(i, _):
        zbuf[pl.ds(i * 16, 16)] = jnp.zeros((16,), _f32)
        return 0
    lax.fori_loop(0, _ZBUF // 16, _fill, 0)

    base0 = wid * _TILE_Z
    zcopies = [pltpu.async_copy(
        zbuf, out_hbm.at[pl.ds(base0 + k * _ZBUF, _ZBUF)], sem)
        for k in range(_TILE_Z // _ZBUF)]
    for cp in zcopies:
        cp.wait()


def _zero_call():
    mesh = plsc.VectorSubcoreMesh(core_axis_name="c", subcore_axis_name="s",
                                  num_cores=NC, num_subcores=NS)
    f = functools.partial(
        pl.kernel, _zero_body,
        out_type=jax.ShapeDtypeStruct((MAT,), _f32),
        mesh=mesh,
        scratch_types=[
            pltpu.VMEM((_ZBUF,), _f32),
            pltpu.SemaphoreType.DMA,
        ],
    )()
    return f()


def _scatter_body(src_hbm, dst_hbm, sco_hbm, mat_hbm,
                  srcv, dstv, valv, i0, i1, i2, i3, sem):
    idx_bufs = (i0, i1, i2, i3)
    wid = lax.axis_index("s") * NC + lax.axis_index("c")
    r0 = wid * _ROWS_PER_TILE
    pltpu.sync_copy(src_hbm.at[pl.ds(r0, _ROWS_PER_TILE)], srcv)
    pltpu.sync_copy(dst_hbm.at[pl.ds(r0, _ROWS_PER_TILE)], dstv)
    pltpu.sync_copy(sco_hbm.at[pl.ds(r0, _ROWS_PER_TILE)], valv)
    for r in range(_ROWS_PER_TILE):
        for k in range(8):
            sv = srcv[r, pl.ds(k * 16, 16)]
            dv = dstv[r, pl.ds(k * 16, 16)]
            idx_bufs[r][pl.ds(k * 16, 16)] = sv * N_PROP + dv
    copies = [pltpu.async_copy(valv.at[r], mat_hbm.at[idx_bufs[r]], sem)
              for r in range(_ROWS_PER_TILE)]
    for cp in copies:
        cp.wait()


def _scatter_call(src2, dst2, sco2, mat_ref):
    mesh = plsc.VectorSubcoreMesh(core_axis_name="c", subcore_axis_name="s",
                                  num_cores=NC, num_subcores=NS)
    f = functools.partial(
        pl.kernel, _scatter_body,
        out_type=(),
        mesh=mesh,
        scratch_types=[
            pltpu.VMEM((_ROWS_PER_TILE, 128), jnp.int32),
            pltpu.VMEM((_ROWS_PER_TILE, 128), jnp.int32),
            pltpu.VMEM((_ROWS_PER_TILE, 128), _f32),
        ] + [pltpu.VMEM((128,), jnp.int32)] * _ROWS_PER_TILE + [
            pltpu.SemaphoreType.DMA,
        ],
    )()
    f(src2, dst2, sco2, mat_ref)


def kernel(visual_feat, boxes, pred_labels, pair_idx, obj_sem_table,
           pos_W1, pos_b1, pos_W2, pos_b2, geo_W, geo_b, vis_W, vis_b,
           fus_g, fus_bln, fus_W, fus_b, cls_g, cls_bln, cls_W, cls_b,
           hyb_W, hyb_b):
    pair_idx = pair_idx.astype(jnp.int32)
    src = pair_idx[:, 0]
    dst = pair_idx[:, 1]
    labels = pred_labels.astype(jnp.int32).reshape(N_PROP, 1)

    gw_ps = geo_W[0:GEO_DIM]
    gw_ss = geo_W[GEO_DIM:GEO_DIM + EMBED_DIM]
    gw_pd = geo_W[GEO_DIM + EMBED_DIM:2 * GEO_DIM + EMBED_DIM]
    gw_sd = geo_W[2 * GEO_DIM + EMBED_DIM:]

    a_tab, b_tab = _prep_call(
        boxes, labels, obj_sem_table, pos_W1, pos_b1.reshape(1, -1),
        pos_W2, pos_b2.reshape(1, -1), gw_ps, gw_ss, gw_pd, gw_sd)

    gs, gd = _gather_call(src, dst, a_tab, b_tab)

    v16 = _vis_call(visual_feat, vis_W.astype(jnp.bfloat16),
                    vis_b.reshape(1, -1))

    mat_ref = jax.new_ref(_zero_call())

    logits_out, scores = _main_call(
        v16, gs, gd, geo_b.reshape(1, -1),
        fus_g[:HIDDEN].reshape(1, -1), fus_g[HIDDEN:].reshape(1, -1),
        fus_bln[:HIDDEN].reshape(1, -1), fus_bln[HIDDEN:].reshape(1, -1),
        fus_W[:HIDDEN].astype(jnp.bfloat16),
        fus_W[HIDDEN:].astype(jnp.bfloat16), fus_b.reshape(1, -1),
        cls_g.reshape(1, -1), cls_bln.reshape(1, -1), cls_W,
        cls_b.reshape(1, -1), hyb_W, hyb_b.reshape(1, 1))

    _scatter_call(src.reshape(128, 128), dst.reshape(128, 128),
                  scores.reshape(128, 128), mat_ref)
    mat = jax.freeze(mat_ref).reshape(N_PROP, N_PROP)
    return (logits_out, mat)


# speedup vs baseline: 1.0419x; 1.0419x over previous
import functools

import jax
import jax.numpy as jnp
from jax import lax
from jax.experimental import pallas as pl
from jax.experimental.pallas import tpu as pltpu
from jax.experimental.pallas import tpu_sc as plsc

N_PROP = 4096
N_PAIRS = 16384
INPUT_DIM = 512
EMBED_DIM = 300
GEO_DIM = 128
HIDDEN = 512
NUM_OBJ = 150
NUM_REL = 50

NC = 2
NS = 16
NW = NC * NS
MAT = N_PROP * N_PROP

_f32 = jnp.float32


def _prep_body(boxes, labels, table, w1, b1, w2, b2,
               gw_ps, gw_ss, gw_pd, gw_sd, a_out, b_out):
    b = boxes[...]
    x1 = b[:, 0:1]
    y1 = b[:, 1:2]
    x2 = b[:, 2:3]
    y2 = b[:, 3:4]
    w = x2 - x1 + 1.0
    h = y2 - y1 + 1.0
    x = x1 + 0.5 * w
    y = y1 + 0.5 * h
    cols = (w, h, x, y, x1, y1, x2, y2, w * h)
    acc = b1[...]
    for j, cj in enumerate(cols):
        acc = acc + cj * w1[j:j + 1, :]
    p1 = jnp.maximum(acc, 0.0)
    pos = jnp.dot(p1, w2[...], preferred_element_type=_f32) + b2[...]
    lab = labels[...]
    onehot = (lab == lax.broadcasted_iota(jnp.int32, (lab.shape[0], NUM_OBJ), 1)
              ).astype(_f32)
    sem = jnp.dot(onehot, table[...], preferred_element_type=_f32)
    rp = jnp.maximum(pos, 0.0)
    rs = jnp.maximum(sem, 0.0)
    a_f = (jnp.dot(rp, gw_ps[...], preferred_element_type=_f32) +
           jnp.dot(rs, gw_ss[...], preferred_element_type=_f32))
    b_f = (jnp.dot(rp, gw_pd[...], preferred_element_type=_f32) +
           jnp.dot(rs, gw_sd[...], preferred_element_type=_f32))
    a_out[...] = _pack_halves(a_f)
    b_out[...] = _pack_halves(b_f)


def _pack_halves(x):
    lo = lax.bitcast_convert_type(x[:, :HIDDEN // 2].astype(jnp.bfloat16),
                                  jnp.uint16).astype(jnp.uint32)
    hi = lax.bitcast_convert_type(x[:, HIDDEN // 2:].astype(jnp.bfloat16),
                                  jnp.uint16).astype(jnp.uint32)
    return lax.bitcast_convert_type((hi << 16) | lo, jnp.int32)


def _unpack_halves(w):
    u = lax.bitcast_convert_type(w, jnp.uint32)
    lo = lax.bitcast_convert_type((u & 0xFFFF).astype(jnp.uint16),
                                  jnp.bfloat16).astype(_f32)
    hi = lax.bitcast_convert_type((u >> 16).astype(jnp.uint16),
                                  jnp.bfloat16).astype(_f32)
    return jnp.concatenate([lo, hi], axis=1)


def _prep_call(boxes, labels, table, w1, b1, w2, b2, gw_ps, gw_ss, gw_pd, gw_sd):
    blk = 1024
    n_blk = N_PROP // blk
    row = lambda i: (i, 0)
    rep = lambda i: (0, 0)
    return pl.pallas_call(
        _prep_body,
        grid=(n_blk,),
        in_specs=[
            pl.BlockSpec((blk, 4), row),
            pl.BlockSpec((blk, 1), row),
            pl.BlockSpec((NUM_OBJ, EMBED_DIM), rep),
            pl.BlockSpec((9, GEO_DIM), rep),
            pl.BlockSpec((1, GEO_DIM), rep),
            pl.BlockSpec((GEO_DIM, GEO_DIM), rep),
            pl.BlockSpec((1, GEO_DIM), rep),
            pl.BlockSpec((GEO_DIM, HIDDEN), rep),
            pl.BlockSpec((EMBED_DIM, HIDDEN), rep),
            pl.BlockSpec((GEO_DIM, HIDDEN), rep),
            pl.BlockSpec((EMBED_DIM, HIDDEN), rep),
        ],
        out_specs=[pl.BlockSpec((blk, HIDDEN // 2), row),
                   pl.BlockSpec((blk, HIDDEN // 2), row)],
        out_shape=[jax.ShapeDtypeStruct((N_PROP, HIDDEN // 2), jnp.int32)] * 2,
    )(boxes, labels, table, w1, b1, w2, b2, gw_ps, gw_ss, gw_pd, gw_sd)


def _gather_body(src_hbm, dst_hbm, a_hbm, b_hbm, gs_hbm, gd_hbm,
                 idx0, idx1, rows0, rows1, is0, is1, os0, os1):
    wid = lax.axis_index("s") * NC + lax.axis_index("c")
    base = wid * (N_PAIRS // NW)
    idx_bufs = (idx0, idx1)
    row_bufs = (rows0, rows1)
    isems = (is0, is1)
    osems = (os0, os1)
    steps = ([(src_hbm, a_hbm, gs_hbm, j) for j in range(4)] +
             [(dst_hbm, b_hbm, gd_hbm, j) for j in range(4)])
    in_d = [None, None]
    out_d = [None, None]

    def fire(t):
        bi = t % 2
        if out_d[bi] is not None:
            out_d[bi].wait()
            out_d[bi] = None
        isrc, tbl, _, j = steps[t]
        pltpu.sync_copy(isrc.at[pl.ds(base + j * 128, 128)], idx_bufs[bi])
        in_d[bi] = pltpu.async_copy(tbl.at[idx_bufs[bi]], row_bufs[bi],
                                    isems[bi])

    fire(0)
    for t in range(len(steps)):
        if t + 1 < len(steps):
            fire(t + 1)
        bi = t % 2
        in_d[bi].wait()
        _, _, out, j = steps[t]
        out_d[bi] = pltpu.async_copy(
            row_bufs[bi], out.at[pl.ds(base + j * 128, 128)], osems[bi])
    for bi in range(2):
        if out_d[bi] is not None:
            out_d[bi].wait()


def _gather_call(src, dst, a, b):
    mesh = plsc.VectorSubcoreMesh(core_axis_name="c", subcore_axis_name="s",
                                  num_cores=NC, num_subcores=NS)
    f = functools.partial(
        pl.kernel, _gather_body,
        out_type=[jax.ShapeDtypeStruct((N_PAIRS, HIDDEN // 2), jnp.int32)] * 2,
        mesh=mesh,
        scratch_types=[
            pltpu.VMEM((128,), jnp.int32),
            pltpu.VMEM((128,), jnp.int32),
            pltpu.VMEM((128, HIDDEN // 2), jnp.int32),
            pltpu.VMEM((128, HIDDEN // 2), jnp.int32),
            pltpu.SemaphoreType.DMA,
            pltpu.SemaphoreType.DMA,
            pltpu.SemaphoreType.DMA,
            pltpu.SemaphoreType.DMA,
        ],
    )()
    return f(src, dst, a, b)


def _main_body(vis, gs, gd, vw, vb, gb, fgv, fgg, fbv, fbg, fwv, fwg, fb,
               cg, cb_ln, cw, cb, hw, hb, out_logits, out_scores):
    v = jnp.dot(jnp.maximum(vis[...], 0.0).astype(jnp.bfloat16), vw[...],
                preferred_element_type=_f32) + vb[...]
    g = _unpack_halves(gs[...]) + _unpack_halves(gd[...]) + gb[...]
    width = 2.0 * HIDDEN
    mu = (jnp.sum(v, axis=1, keepdims=True) +
          jnp.sum(g, axis=1, keepdims=True)) / width
    dv = v - mu
    dg = g - mu
    var = (jnp.sum(dv * dv, axis=1, keepdims=True) +
           jnp.sum(dg * dg, axis=1, keepdims=True)) / width
    inv = lax.rsqrt(var + 1e-5)
    nv = jnp.maximum(dv * inv * fgv[...] + fbv[...], 0.0).astype(jnp.bfloat16)
    ng = jnp.maximum(dg * inv * fgg[...] + fbg[...], 0.0).astype(jnp.bfloat16)
    h = (jnp.dot(nv, fwv[...], preferred_element_type=_f32) +
         jnp.dot(ng, fwg[...], preferred_element_type=_f32) + fb[...])
    mu2 = jnp.mean(h, axis=1, keepdims=True)
    dh = h - mu2
    var2 = jnp.mean(dh * dh, axis=1, keepdims=True)
    n2 = jnp.maximum(dh * lax.rsqrt(var2 + 1e-5) * cg[...] + cb_ln[...], 0.0)
    logits = jnp.dot(n2, cw[...], preferred_element_type=_f32) + cb[...]
    bin_l = jnp.dot(logits, hw[...], preferred_element_type=_f32) + hb[...]
    out_logits[...] = jnp.concatenate([logits, bin_l], axis=1)
    out_scores[...] = jax.nn.sigmoid(bin_l)


def _main_call(vis, gs, gd, vw, vb, gb, fgv, fgg, fbv, fbg, fwv, fwg, fb,
               cg, cb_ln, cw, cb, hw, hb):
    blk = 1024
    n_blk = N_PAIRS // blk
    row = lambda i: (i, 0)
    rep = lambda i: (0, 0)
    wide = lambda shape: pl.BlockSpec(shape, rep)
    return pl.pallas_call(
        _main_body,
        grid=(n_blk,),
        in_specs=[
            pl.BlockSpec((blk, INPUT_DIM), row),
            pl.BlockSpec((blk, HIDDEN // 2), row),
            pl.BlockSpec((blk, HIDDEN // 2), row),
            wide((INPUT_DIM, HIDDEN)),
            wide((1, HIDDEN)),
            wide((1, HIDDEN)),
            wide((1, HIDDEN)),
            wide((1, HIDDEN)),
            wide((1, HIDDEN)),
            wide((1, HIDDEN)),
            wide((HIDDEN, HIDDEN)),
            wide((HIDDEN, HIDDEN)),
            wide((1, HIDDEN)),
            wide((1, HIDDEN)),
            wide((1, HIDDEN)),
            wide((HIDDEN, NUM_REL)),
            wide((1, NUM_REL)),
            wide((NUM_REL, 1)),
            wide((1, 1)),
        ],
        out_specs=[pl.BlockSpec((blk, NUM_REL + 1), row),
                   pl.BlockSpec((blk, 1), row)],
        out_shape=[jax.ShapeDtypeStruct((N_PAIRS, NUM_REL + 1), _f32),
                   jax.ShapeDtypeStruct((N_PAIRS, 1), _f32)],
    )(vis, gs, gd, vw, vb, gb, fgv, fgg, fbv, fbg, fwv, fwg, fb,
      cg, cb_ln, cw, cb, hw, hb)


_TILE_Z = MAT // NW
_ZBUF = 32768
_ROWS_PER_TILE = 4


def _zero_body(out_hbm, zbuf, sem):
    wid = lax.axis_index("s") * NC + lax.axis_index("c")

    def _fill(i, _):
        zbuf[pl.ds(i * 16, 16)] = jnp.zeros((16,), _f32)
        return 0
    lax.fori_loop(0, _ZBUF // 16, _fill, 0)

    base0 = wid * _TILE_Z
    zcopies = [pltpu.async_copy(
        zbuf, out_hbm.at[pl.ds(base0 + k * _ZBUF, _ZBUF)], sem)
        for k in range(_TILE_Z // _ZBUF)]
    for cp in zcopies:
        cp.wait()


def _zero_call():
    mesh = plsc.VectorSubcoreMesh(core_axis_name="c", subcore_axis_name="s",
                                  num_cores=NC, num_subcores=NS)
    f = functools.partial(
        pl.kernel, _zero_body,
        out_type=jax.ShapeDtypeStruct((MAT,), _f32),
        mesh=mesh,
        scratch_types=[
            pltpu.VMEM((_ZBUF,), _f32),
            pltpu.SemaphoreType.DMA,
        ],
    )()
    return f()


def _scatter_body(src_hbm, dst_hbm, sco_hbm, mat_hbm,
                  srcv, dstv, valv, i0, i1, i2, i3, sem):
    idx_bufs = (i0, i1, i2, i3)
    wid = lax.axis_index("s") * NC + lax.axis_index("c")
    r0 = wid * _ROWS_PER_TILE
    pltpu.sync_copy(src_hbm.at[pl.ds(r0, _ROWS_PER_TILE)], srcv)
    pltpu.sync_copy(dst_hbm.at[pl.ds(r0, _ROWS_PER_TILE)], dstv)
    pltpu.sync_copy(sco_hbm.at[pl.ds(r0, _ROWS_PER_TILE)], valv)
    for r in range(_ROWS_PER_TILE):
        for k in range(8):
            sv = srcv[r, pl.ds(k * 16, 16)]
            dv = dstv[r, pl.ds(k * 16, 16)]
            idx_bufs[r][pl.ds(k * 16, 16)] = sv * N_PROP + dv
    copies = [pltpu.async_copy(valv.at[r], mat_hbm.at[idx_bufs[r]], sem)
              for r in range(_ROWS_PER_TILE)]
    for cp in copies:
        cp.wait()


def _scatter_call(src2, dst2, sco2, mat_ref):
    mesh = plsc.VectorSubcoreMesh(core_axis_name="c", subcore_axis_name="s",
                                  num_cores=NC, num_subcores=NS)
    f = functools.partial(
        pl.kernel, _scatter_body,
        out_type=(),
        mesh=mesh,
        scratch_types=[
            pltpu.VMEM((_ROWS_PER_TILE, 128), jnp.int32),
            pltpu.VMEM((_ROWS_PER_TILE, 128), jnp.int32),
            pltpu.VMEM((_ROWS_PER_TILE, 128), _f32),
        ] + [pltpu.VMEM((128,), jnp.int32)] * _ROWS_PER_TILE + [
            pltpu.SemaphoreType.DMA,
        ],
    )()
    f(src2, dst2, sco2, mat_ref)


def kernel(visual_feat, boxes, pred_labels, pair_idx, obj_sem_table,
           pos_W1, pos_b1, pos_W2, pos_b2, geo_W, geo_b, vis_W, vis_b,
           fus_g, fus_bln, fus_W, fus_b, cls_g, cls_bln, cls_W, cls_b,
           hyb_W, hyb_b):
    pair_idx = pair_idx.astype(jnp.int32)
    src = pair_idx[:, 0]
    dst = pair_idx[:, 1]
    labels = pred_labels.astype(jnp.int32).reshape(N_PROP, 1)

    gw_ps = geo_W[0:GEO_DIM]
    gw_ss = geo_W[GEO_DIM:GEO_DIM + EMBED_DIM]
    gw_pd = geo_W[GEO_DIM + EMBED_DIM:2 * GEO_DIM + EMBED_DIM]
    gw_sd = geo_W[2 * GEO_DIM + EMBED_DIM:]

    a_tab, b_tab = _prep_call(
        boxes, labels, obj_sem_table, pos_W1, pos_b1.reshape(1, -1),
        pos_W2, pos_b2.reshape(1, -1), gw_ps, gw_ss, gw_pd, gw_sd)

    gs, gd = _gather_call(src, dst, a_tab, b_tab)

    mat_ref = jax.new_ref(_zero_call())

    logits_out, scores = _main_call(
        visual_feat, gs, gd, vis_W.astype(jnp.bfloat16),
        vis_b.reshape(1, -1), geo_b.reshape(1, -1),
        fus_g[:HIDDEN].reshape(1, -1), fus_g[HIDDEN:].reshape(1, -1),
        fus_bln[:HIDDEN].reshape(1, -1), fus_bln[HIDDEN:].reshape(1, -1),
        fus_W[:HIDDEN].astype(jnp.bfloat16),
        fus_W[HIDDEN:].astype(jnp.bfloat16), fus_b.reshape(1, -1),
        cls_g.reshape(1, -1), cls_bln.reshape(1, -1), cls_W,
        cls_b.reshape(1, -1), hyb_W, hyb_b.reshape(1, 1))

    _scatter_call(src.reshape(128, 128), dst.reshape(128, 128),
                  scores.reshape(128, 128), mat_ref)
    mat = jax.freeze(mat_ref).reshape(N_PROP, N_PROP)
    return (logits_out, mat)


# speedup vs baseline: 1.0603x; 1.0177x over previous
import functools

import jax
import jax.numpy as jnp
from jax import lax
from jax.experimental import pallas as pl
from jax.experimental.pallas import tpu as pltpu
from jax.experimental.pallas import tpu_sc as plsc

N_PROP = 4096
N_PAIRS = 16384
INPUT_DIM = 512
EMBED_DIM = 300
GEO_DIM = 128
HIDDEN = 512
NUM_OBJ = 150
NUM_REL = 50

NC = 2
NS = 16
NW = NC * NS
MAT = N_PROP * N_PROP

_f32 = jnp.float32


def _prep_body(boxes, labels, table, w1, b1, w2, b2,
               gw_ps, gw_ss, gw_pd, gw_sd, a_out, b_out):
    b = boxes[...]
    x1 = b[:, 0:1]
    y1 = b[:, 1:2]
    x2 = b[:, 2:3]
    y2 = b[:, 3:4]
    w = x2 - x1 + 1.0
    h = y2 - y1 + 1.0
    x = x1 + 0.5 * w
    y = y1 + 0.5 * h
    cols = (w, h, x, y, x1, y1, x2, y2, w * h)
    acc = b1[...]
    for j, cj in enumerate(cols):
        acc = acc + cj * w1[j:j + 1, :]
    p1 = jnp.maximum(acc, 0.0)
    pos = jnp.dot(p1, w2[...], preferred_element_type=_f32) + b2[...]
    lab = labels[...]
    onehot = (lab == lax.broadcasted_iota(jnp.int32, (lab.shape[0], NUM_OBJ), 1)
              ).astype(_f32)
    sem = jnp.dot(onehot, table[...], preferred_element_type=_f32)
    rp = jnp.maximum(pos, 0.0)
    rs = jnp.maximum(sem, 0.0)
    a_f = (jnp.dot(rp, gw_ps[...], preferred_element_type=_f32) +
           jnp.dot(rs, gw_ss[...], preferred_element_type=_f32))
    b_f = (jnp.dot(rp, gw_pd[...], preferred_element_type=_f32) +
           jnp.dot(rs, gw_sd[...], preferred_element_type=_f32))
    a_out[...] = _pack_halves(a_f)
    b_out[...] = _pack_halves(b_f)


def _pack_halves(x):
    lo = lax.bitcast_convert_type(x[:, :HIDDEN // 2].astype(jnp.bfloat16),
                                  jnp.uint16).astype(jnp.uint32)
    hi = lax.bitcast_convert_type(x[:, HIDDEN // 2:].astype(jnp.bfloat16),
                                  jnp.uint16).astype(jnp.uint32)
    return lax.bitcast_convert_type((hi << 16) | lo, jnp.int32)


def _unpack_halves(w):
    u = lax.bitcast_convert_type(w, jnp.uint32)
    lo = lax.bitcast_convert_type((u & 0xFFFF).astype(jnp.uint16),
                                  jnp.bfloat16).astype(_f32)
    hi = lax.bitcast_convert_type((u >> 16).astype(jnp.uint16),
                                  jnp.bfloat16).astype(_f32)
    return jnp.concatenate([lo, hi], axis=1)


def _prep_call(boxes, labels, table, w1, b1, w2, b2, gw_ps, gw_ss, gw_pd, gw_sd):
    blk = 1024
    n_blk = N_PROP // blk
    row = lambda i: (i, 0)
    rep = lambda i: (0, 0)
    return pl.pallas_call(
        _prep_body,
        grid=(n_blk,),
        in_specs=[
            pl.BlockSpec((blk, 4), row),
            pl.BlockSpec((blk, 1), row),
            pl.BlockSpec((NUM_OBJ, EMBED_DIM), rep),
            pl.BlockSpec((9, GEO_DIM), rep),
            pl.BlockSpec((1, GEO_DIM), rep),
            pl.BlockSpec((GEO_DIM, GEO_DIM), rep),
            pl.BlockSpec((1, GEO_DIM), rep),
            pl.BlockSpec((GEO_DIM, HIDDEN), rep),
            pl.BlockSpec((EMBED_DIM, HIDDEN), rep),
            pl.BlockSpec((GEO_DIM, HIDDEN), rep),
            pl.BlockSpec((EMBED_DIM, HIDDEN), rep),
        ],
        out_specs=[pl.BlockSpec((blk, HIDDEN // 2), row),
                   pl.BlockSpec((blk, HIDDEN // 2), row)],
        out_shape=[jax.ShapeDtypeStruct((N_PROP, HIDDEN // 2), jnp.int32)] * 2,
    )(boxes, labels, table, w1, b1, w2, b2, gw_ps, gw_ss, gw_pd, gw_sd)


def _gather_body(src_hbm, dst_hbm, a_hbm, b_hbm, gs_hbm, gd_hbm,
                 idx0, idx1, rows0, rows1, is0, is1, os0, os1):
    wid = lax.axis_index("s") * NC + lax.axis_index("c")
    base = wid * (N_PAIRS // NW)
    idx_bufs = (idx0, idx1)
    row_bufs = (rows0, rows1)
    isems = (is0, is1)
    osems = (os0, os1)
    steps = ([(src_hbm, a_hbm, gs_hbm, j) for j in range(4)] +
             [(dst_hbm, b_hbm, gd_hbm, j) for j in range(4)])
    in_d = [None, None]
    out_d = [None, None]

    def fire(t):
        bi = t % 2
        if out_d[bi] is not None:
            out_d[bi].wait()
            out_d[bi] = None
        isrc, tbl, _, j = steps[t]
        pltpu.sync_copy(isrc.at[pl.ds(base + j * 128, 128)], idx_bufs[bi])
        in_d[bi] = pltpu.async_copy(tbl.at[idx_bufs[bi]], row_bufs[bi],
                                    isems[bi])

    fire(0)
    for t in range(len(steps)):
        if t + 1 < len(steps):
            fire(t + 1)
        bi = t % 2
        in_d[bi].wait()
        _, _, out, j = steps[t]
        out_d[bi] = pltpu.async_copy(
            row_bufs[bi], out.at[pl.ds(base + j * 128, 128)], osems[bi])
    for bi in range(2):
        if out_d[bi] is not None:
            out_d[bi].wait()


def _gather_call(src, dst, a, b):
    mesh = plsc.VectorSubcoreMesh(core_axis_name="c", subcore_axis_name="s",
                                  num_cores=NC, num_subcores=NS)
    f = functools.partial(
        pl.kernel, _gather_body,
        out_type=[jax.ShapeDtypeStruct((N_PAIRS, HIDDEN // 2), jnp.int32)] * 2,
        mesh=mesh,
        scratch_types=[
            pltpu.VMEM((128,), jnp.int32),
            pltpu.VMEM((128,), jnp.int32),
            pltpu.VMEM((128, HIDDEN // 2), jnp.int32),
            pltpu.VMEM((128, HIDDEN // 2), jnp.int32),
            pltpu.SemaphoreType.DMA,
            pltpu.SemaphoreType.DMA,
            pltpu.SemaphoreType.DMA,
            pltpu.SemaphoreType.DMA,
        ],
    )()
    return f(src, dst, a, b)


def _main_body(vis, gs, gd, vw, vb, gb, fgv, fgg, fbv, fbg, fwv, fwg, fb,
               cg, cb_ln, cw, cb, hw, hb, out_logits, out_scores):
    v = jnp.dot(jnp.maximum(vis[...], 0.0).astype(jnp.bfloat16), vw[...],
                preferred_element_type=_f32) + vb[...]
    g = _unpack_halves(gs[...]) + _unpack_halves(gd[...]) + gb[...]
    width = 2.0 * HIDDEN
    mu = (jnp.sum(v, axis=1, keepdims=True) +
          jnp.sum(g, axis=1, keepdims=True)) / width
    dv = v - mu
    dg = g - mu
    var = (jnp.sum(dv * dv, axis=1, keepdims=True) +
           jnp.sum(dg * dg, axis=1, keepdims=True)) / width
    inv = lax.rsqrt(var + 1e-5)
    nv = jnp.maximum(dv * inv * fgv[...] + fbv[...], 0.0).astype(jnp.bfloat16)
    ng = jnp.maximum(dg * inv * fgg[...] + fbg[...], 0.0).astype(jnp.bfloat16)
    h = (jnp.dot(nv, fwv[...], preferred_element_type=_f32) +
         jnp.dot(ng, fwg[...], preferred_element_type=_f32) + fb[...])
    mu2 = jnp.mean(h, axis=1, keepdims=True)
    dh = h - mu2
    var2 = jnp.mean(dh * dh, axis=1, keepdims=True)
    n2 = jnp.maximum(dh * lax.rsqrt(var2 + 1e-5) * cg[...] + cb_ln[...], 0.0)
    logits = jnp.dot(n2, cw[...], preferred_element_type=_f32) + cb[...]
    bin_l = jnp.dot(logits, hw[...], preferred_element_type=_f32) + hb[...]
    out_logits[...] = jnp.concatenate([logits, bin_l], axis=1)
    out_scores[...] = jax.nn.sigmoid(bin_l)


def _main_call(vis, gs, gd, vw, vb, gb, fgv, fgg, fbv, fbg, fwv, fwg, fb,
               cg, cb_ln, cw, cb, hw, hb):
    blk = 2048
    n_blk = N_PAIRS // blk
    row = lambda i: (i, 0)
    rep = lambda i: (0, 0)
    wide = lambda shape: pl.BlockSpec(shape, rep)
    return pl.pallas_call(
        _main_body,
        grid=(n_blk,),
        in_specs=[
            pl.BlockSpec((blk, INPUT_DIM), row),
            pl.BlockSpec((blk, HIDDEN // 2), row),
            pl.BlockSpec((blk, HIDDEN // 2), row),
            wide((INPUT_DIM, HIDDEN)),
            wide((1, HIDDEN)),
            wide((1, HIDDEN)),
            wide((1, HIDDEN)),
            wide((1, HIDDEN)),
            wide((1, HIDDEN)),
            wide((1, HIDDEN)),
            wide((HIDDEN, HIDDEN)),
            wide((HIDDEN, HIDDEN)),
            wide((1, HIDDEN)),
            wide((1, HIDDEN)),
            wide((1, HIDDEN)),
            wide((HIDDEN, NUM_REL)),
            wide((1, NUM_REL)),
            wide((NUM_REL, 1)),
            wide((1, 1)),
        ],
        out_specs=[pl.BlockSpec((blk, NUM_REL + 1), row),
                   pl.BlockSpec((blk, 1), row)],
        out_shape=[jax.ShapeDtypeStruct((N_PAIRS, NUM_REL + 1), _f32),
                   jax.ShapeDtypeStruct((N_PAIRS, 1), _f32)],
    )(vis, gs, gd, vw, vb, gb, fgv, fgg, fbv, fbg, fwv, fwg, fb,
      cg, cb_ln, cw, cb, hw, hb)


_TILE_Z = MAT // NW
_ZBUF = 32768
_ROWS_PER_TILE = 4


def _zero_body(out_hbm, zbuf, sem):
    wid = lax.axis_index("s") * NC + lax.axis_index("c")

    def _fill(i, _):
        zbuf[pl.ds(i * 16, 16)] = jnp.zeros((16,), _f32)
        return 0
    lax.fori_loop(0, _ZBUF // 16, _fill, 0)

    base0 = wid * _TILE_Z
    zcopies = [pltpu.async_copy(
        zbuf, out_hbm.at[pl.ds(base0 + k * _ZBUF, _ZBUF)], sem)
        for k in range(_TILE_Z // _ZBUF)]
    for cp in zcopies:
        cp.wait()


def _zero_call():
    mesh = plsc.VectorSubcoreMesh(core_axis_name="c", subcore_axis_name="s",
                                  num_cores=NC, num_subcores=NS)
    f = functools.partial(
        pl.kernel, _zero_body,
        out_type=jax.ShapeDtypeStruct((MAT,), _f32),
        mesh=mesh,
        scratch_types=[
            pltpu.VMEM((_ZBUF,), _f32),
            pltpu.SemaphoreType.DMA,
        ],
    )()
    return f()


def _scatter_body(src_hbm, dst_hbm, sco_hbm, mat_hbm,
                  srcv, dstv, valv, i0, i1, i2, i3, sem):
    idx_bufs = (i0, i1, i2, i3)
    wid = lax.axis_index("s") * NC + lax.axis_index("c")
    r0 = wid * _ROWS_PER_TILE
    pltpu.sync_copy(src_hbm.at[pl.ds(r0, _ROWS_PER_TILE)], srcv)
    pltpu.sync_copy(dst_hbm.at[pl.ds(r0, _ROWS_PER_TILE)], dstv)
    pltpu.sync_copy(sco_hbm.at[pl.ds(r0, _ROWS_PER_TILE)], valv)
    for r in range(_ROWS_PER_TILE):
        for k in range(8):
            sv = srcv[r, pl.ds(k * 16, 16)]
            dv = dstv[r, pl.ds(k * 16, 16)]
            idx_bufs[r][pl.ds(k * 16, 16)] = sv * N_PROP + dv
    copies = [pltpu.async_copy(valv.at[r], mat_hbm.at[idx_bufs[r]], sem)
              for r in range(_ROWS_PER_TILE)]
    for cp in copies:
        cp.wait()


def _scatter_call(src2, dst2, sco2, mat_ref):
    mesh = plsc.VectorSubcoreMesh(core_axis_name="c", subcore_axis_name="s",
                                  num_cores=NC, num_subcores=NS)
    f = functools.partial(
        pl.kernel, _scatter_body,
        out_type=(),
        mesh=mesh,
        scratch_types=[
            pltpu.VMEM((_ROWS_PER_TILE, 128), jnp.int32),
            pltpu.VMEM((_ROWS_PER_TILE, 128), jnp.int32),
            pltpu.VMEM((_ROWS_PER_TILE, 128), _f32),
        ] + [pltpu.VMEM((128,), jnp.int32)] * _ROWS_PER_TILE + [
            pltpu.SemaphoreType.DMA,
        ],
    )()
    f(src2, dst2, sco2, mat_ref)


def kernel(visual_feat, boxes, pred_labels, pair_idx, obj_sem_table,
           pos_W1, pos_b1, pos_W2, pos_b2, geo_W, geo_b, vis_W, vis_b,
           fus_g, fus_bln, fus_W, fus_b, cls_g, cls_bln, cls_W, cls_b,
           hyb_W, hyb_b):
    pair_idx = pair_idx.astype(jnp.int32)
    src = pair_idx[:, 0]
    dst = pair_idx[:, 1]
    labels = pred_labels.astype(jnp.int32).reshape(N_PROP, 1)

    gw_ps = geo_W[0:GEO_DIM]
    gw_ss = geo_W[GEO_DIM:GEO_DIM + EMBED_DIM]
    gw_pd = geo_W[GEO_DIM + EMBED_DIM:2 * GEO_DIM + EMBED_DIM]
    gw_sd = geo_W[2 * GEO_DIM + EMBED_DIM:]

    a_tab, b_tab = _prep_call(
        boxes, labels, obj_sem_table, pos_W1, pos_b1.reshape(1, -1),
        pos_W2, pos_b2.reshape(1, -1), gw_ps, gw_ss, gw_pd, gw_sd)

    gs, gd = _gather_call(src, dst, a_tab, b_tab)

    mat_ref = jax.new_ref(_zero_call())

    logits_out, scores = _main_call(
        visual_feat, gs, gd, vis_W.astype(jnp.bfloat16),
        vis_b.reshape(1, -1), geo_b.reshape(1, -1),
        fus_g[:HIDDEN].reshape(1, -1), fus_g[HIDDEN:].reshape(1, -1),
        fus_bln[:HIDDEN].reshape(1, -1), fus_bln[HIDDEN:].reshape(1, -1),
        fus_W[:HIDDEN].astype(jnp.bfloat16),
        fus_W[HIDDEN:].astype(jnp.bfloat16), fus_b.reshape(1, -1),
        cls_g.reshape(1, -1), cls_bln.reshape(1, -1), cls_W,
        cls_b.reshape(1, -1), hyb_W, hyb_b.reshape(1, 1))

    _scatter_call(src.reshape(128, 128), dst.reshape(128, 128),
                  scores.reshape(128, 128), mat_ref)
    mat = jax.freeze(mat_ref).reshape(N_PROP, N_PROP)
    return (logits_out, mat)
